# baseline (speedup 1.0000x reference)
import jax
import jax.numpy as jnp
from jax.experimental import pallas as pl
from jax.experimental.pallas import tpu as pltpu

_CODE_DIM = 32
_NUM_CODES = 8192
_TOKEN_BLOCK = 8192


def _l2norm(x):
    return x / jnp.maximum(
        jnp.sqrt(jnp.sum(x * x, axis=1, keepdims=True)), 1e-8)


def _vq_kernel(x_ref, cb_ref, out_ref, cbn_ref):
    @pl.when(pl.program_id(0) == 0)
    def _():
        cbn_ref[...] = _l2norm(cb_ref[...])

    xn = _l2norm(x_ref[...])
    cbn = cbn_ref[...]
    h = _TOKEN_BLOCK // 16
    for k in range(16):
        logits = jax.lax.dot_general(
            xn[k * h:(k + 1) * h], cbn, (((1,), (1,)), ((), ())),
            preferred_element_type=jnp.float32)
        out_ref[0, 0, k * h:(k + 1) * h] = jnp.argmax(
            logits, axis=1).astype(jnp.int32)


def kernel(z_e, codebook):
    b, t, d = z_e.shape
    n_tokens = b * t
    flat = z_e.reshape(n_tokens, d)
    n_blocks = n_tokens // _TOKEN_BLOCK

    out = pl.pallas_call(
        _vq_kernel,
        grid=(n_blocks,),
        in_specs=[
            pl.BlockSpec((_TOKEN_BLOCK, _CODE_DIM), lambda i: (i, 0)),
            pl.BlockSpec((_NUM_CODES, _CODE_DIM), lambda i: (0, 0)),
        ],
        out_specs=pl.BlockSpec((1, 1, _TOKEN_BLOCK), lambda i: (i, 0, 0)),
        out_shape=jax.ShapeDtypeStruct((n_blocks, 1, _TOKEN_BLOCK), jnp.int32),
        scratch_shapes=[pltpu.VMEM((_NUM_CODES, _CODE_DIM), jnp.float32)],
    )(flat, codebook)
    return out.reshape(b, t)


# speedup vs baseline: 1.0500x; 1.0500x over previous
import jax
import jax.numpy as jnp
from jax.experimental import pallas as pl
from jax.experimental.pallas import tpu as pltpu

_CODE_DIM = 32
_NUM_CODES = 8192
_TOKEN_BLOCK = 4096


def _l2norm(x):
    return x / jnp.maximum(
        jnp.sqrt(jnp.sum(x * x, axis=1, keepdims=True)), 1e-8)


def _vq_kernel(x_ref, cb_ref, out_ref, cbn_ref):
    @pl.when(pl.program_id(0) == 0)
    def _():
        cbn_ref[...] = _l2norm(cb_ref[...])

    xn = _l2norm(x_ref[...])
    cbn = cbn_ref[...]
    h = _TOKEN_BLOCK // 16
    for k in range(16):
        logits = jax.lax.dot_general(
            xn[k * h:(k + 1) * h], cbn, (((1,), (1,)), ((), ())),
            preferred_element_type=jnp.float32)
        out_ref[0, 0, k * h:(k + 1) * h] = jnp.argmax(
            logits, axis=1).astype(jnp.int32)


def kernel(z_e, codebook):
    b, t, d = z_e.shape
    n_tokens = b * t
    flat = z_e.reshape(n_tokens, d)
    n_blocks = n_tokens // _TOKEN_BLOCK

    out = pl.pallas_call(
        _vq_kernel,
        grid=(n_blocks,),
        in_specs=[
            pl.BlockSpec((_TOKEN_BLOCK, _CODE_DIM), lambda i: (i, 0)),
            pl.BlockSpec((_NUM_CODES, _CODE_DIM), lambda i: (0, 0)),
        ],
        out_specs=pl.BlockSpec((1, 1, _TOKEN_BLOCK), lambda i: (i, 0, 0)),
        out_shape=jax.ShapeDtypeStruct((n_blocks, 1, _TOKEN_BLOCK), jnp.int32),
        scratch_shapes=[pltpu.VMEM((_NUM_CODES, _CODE_DIM), jnp.float32)],
    )(flat, codebook)
    return out.reshape(b, t)


# speedup vs baseline: 1.0547x; 1.0045x over previous
import jax
import jax.numpy as jnp
from jax.experimental import pallas as pl
from jax.experimental.pallas import tpu as pltpu

_CODE_DIM = 32
_NUM_CODES = 8192
_TOKEN_BLOCK = 4096


def _l2norm(x):
    return x / jnp.maximum(
        jnp.sqrt(jnp.sum(x * x, axis=1, keepdims=True)), 1e-8)


def _vq_kernel(x_ref, cb_ref, out_ref, cbn_ref):
    @pl.when(pl.program_id(0) == 0)
    def _():
        cbn_ref[...] = _l2norm(cb_ref[...])

    xn = _l2norm(x_ref[...])
    cbn = cbn_ref[...]
    h = _TOKEN_BLOCK // 32
    for k in range(32):
        logits = jax.lax.dot_general(
            xn[k * h:(k + 1) * h], cbn, (((1,), (1,)), ((), ())),
            preferred_element_type=jnp.float32)
        out_ref[0, 0, k * h:(k + 1) * h] = jnp.argmax(
            logits, axis=1).astype(jnp.int32)


def kernel(z_e, codebook):
    b, t, d = z_e.shape
    n_tokens = b * t
    flat = z_e.reshape(n_tokens, d)
    n_blocks = n_tokens // _TOKEN_BLOCK

    out = pl.pallas_call(
        _vq_kernel,
        grid=(n_blocks,),
        in_specs=[
            pl.BlockSpec((_TOKEN_BLOCK, _CODE_DIM), lambda i: (i, 0)),
            pl.BlockSpec((_NUM_CODES, _CODE_DIM), lambda i: (0, 0)),
        ],
        out_specs=pl.BlockSpec((1, 1, _TOKEN_BLOCK), lambda i: (i, 0, 0)),
        out_shape=jax.ShapeDtypeStruct((n_blocks, 1, _TOKEN_BLOCK), jnp.int32),
        scratch_shapes=[pltpu.VMEM((_NUM_CODES, _CODE_DIM), jnp.float32)],
    )(flat, codebook)
    return out.reshape(b, t)
